# Initial kernel scaffold; baseline (speedup 1.0000x reference)
import functools

import jax
import jax.numpy as jnp
from jax import lax
from jax.experimental import pallas as pl
from jax.experimental.pallas import tpu as pltpu
from jax.experimental.pallas import tpu_sc as plsc

VOCAB = 100000
DIM = 128
NC = 2
NS = 16
NW = NC * NS
ROWS_PER_W = VOCAB // NW
CHUNK = 125
NCHUNK = ROWS_PER_W // CHUNK
NLG = DIM // 16
RUNROLL = 5

_mesh = plsc.VectorSubcoreMesh(
    core_axis_name="c", subcore_axis_name="s", num_cores=NC, num_subcores=NS
)


@functools.partial(
    pl.kernel,
    out_type=jax.ShapeDtypeStruct((4, NW, DIM), jnp.float32),
    mesh=_mesh,
    scratch_types=[
        pltpu.VMEM((2, CHUNK, DIM), jnp.float32),
        pltpu.VMEM((4, DIM), jnp.float32),
        pltpu.SemaphoreType.DMA((2,)),
    ],
)
def _colsums(t0, t1, t2, t3, out_hbm, buf, acc, sems):
    wid = lax.axis_index("s") * NC + lax.axis_index("c")
    base = wid * ROWS_PER_W

    for t, tref in enumerate((t0, t1, t2, t3)):
        pltpu.make_async_copy(
            tref.at[pl.ds(base, CHUNK)], buf.at[0], sems.at[0]
        ).start()

        def chunk_body(i, accs, tref=tref):
            parity = lax.rem(i, 2)
            nxt = 1 - parity
            pltpu.make_async_copy(
                tref.at[pl.ds(base + i * CHUNK, CHUNK)],
                buf.at[parity],
                sems.at[parity],
            ).wait()

            @pl.when(i + 1 < NCHUNK)
            def _():
                pltpu.make_async_copy(
                    tref.at[pl.ds(base + (i + 1) * CHUNK, CHUNK)],
                    buf.at[nxt],
                    sems.at[nxt],
                ).start()

            def row_body(r, accs):
                r0 = r * RUNROLL
                accs = list(accs)
                for u in range(RUNROLL):
                    for k in range(NLG):
                        accs[k] = accs[k] + buf[parity, r0 + u, pl.ds(k * 16, 16)]
                return tuple(accs)

            return lax.fori_loop(0, CHUNK // RUNROLL, row_body, accs)

        accs = tuple(jnp.zeros((16,), jnp.float32) for _ in range(NLG))
        accs = lax.fori_loop(0, NCHUNK, chunk_body, accs)
        for k in range(NLG):
            acc[t, pl.ds(k * 16, 16)] = accs[k]

    for t in range(4):
        pltpu.sync_copy(acc.at[t], out_hbm.at[t, wid])


def _finish_body(p_ref, r0_ref, w_ref, lmd_ref, out_ref):
    p = p_ref[...]
    r0 = r0_ref[...]
    inv = 1.0 / (VOCAB - 1)
    s = [
        jnp.sum(p[t * NW:(t + 1) * NW], axis=0, keepdims=True)
        - r0[t:t + 1]
        for t in range(4)
    ]
    du = (s[0] - s[2]) * inv
    dh = (s[1] - s[3]) * inv
    vec_sum = jnp.sum(du * du + dh * dh)
    w = w_ref[...]
    eye = (
        lax.broadcasted_iota(jnp.int32, (DIM, DIM), 0)
        == lax.broadcasted_iota(jnp.int32, (DIM, DIM), 1)
    ).astype(jnp.float32)
    wi = w - eye
    out_ref[0, 0] = jnp.sum(wi * wi) + DIM * lmd_ref[0, 0] * vec_sum


_finish = pl.pallas_call(
    _finish_body,
    out_shape=jax.ShapeDtypeStruct((1, 1), jnp.float32),
)


def kernel(emb1_u, emb1_h, emb2_u, emb2_h, W, lmd):
    partials = _colsums(emb1_u, emb1_h, emb2_u, emb2_h)
    row0 = jnp.stack([emb1_u[0], emb1_h[0], emb2_u[0], emb2_h[0]])
    lmd_arr = jnp.asarray(lmd, jnp.float32).reshape(1, 1)
    out = _finish(partials.reshape(4 * NW, DIM), row0, W, lmd_arr)
    return out[0, 0]


# speedup vs baseline: 5.5975x; 5.5975x over previous
import functools

import jax
import jax.numpy as jnp
from jax import lax
from jax.experimental import pallas as pl
from jax.experimental.pallas import tpu as pltpu
from jax.experimental.pallas import tpu_sc as plsc

VOCAB = 100000
DIM = 128
NC = 2
NS = 16
NW = NC * NS
ROWS_PER_W = VOCAB // NW
CHUNK = 125
NCHUNK = ROWS_PER_W // CHUNK
NLG = DIM // 16
RUNROLL = 5

_mesh = plsc.VectorSubcoreMesh(
    core_axis_name="c", subcore_axis_name="s", num_cores=NC, num_subcores=NS
)


@functools.partial(
    pl.kernel,
    out_type=jax.ShapeDtypeStruct((4, NW, DIM), jnp.float32),
    mesh=_mesh,
    scratch_types=[
        pltpu.VMEM((2, CHUNK, DIM), jnp.float32),
        pltpu.VMEM((4, DIM), jnp.float32),
        pltpu.SemaphoreType.DMA((2,)),
    ],
    compiler_params=pltpu.CompilerParams(use_tc_tiling_on_sc=False),
)
def _colsums(t0, t1, t2, t3, out_hbm, buf, acc, sems):
    wid = lax.axis_index("s") * NC + lax.axis_index("c")
    base = wid * ROWS_PER_W

    for t, tref in enumerate((t0, t1, t2, t3)):
        pltpu.make_async_copy(
            tref.at[pl.ds(base, CHUNK)], buf.at[0], sems.at[0]
        ).start()

        def chunk_body(i, accs, tref=tref):
            parity = lax.rem(i, 2)
            nxt = 1 - parity
            pltpu.make_async_copy(
                tref.at[pl.ds(base + i * CHUNK, CHUNK)],
                buf.at[parity],
                sems.at[parity],
            ).wait()

            @pl.when(i + 1 < NCHUNK)
            def _():
                pltpu.make_async_copy(
                    tref.at[pl.ds(base + (i + 1) * CHUNK, CHUNK)],
                    buf.at[nxt],
                    sems.at[nxt],
                ).start()

            def row_body(r, accs):
                r0 = r * RUNROLL
                accs = list(accs)
                for u in range(RUNROLL):
                    for k in range(NLG):
                        accs[k] = accs[k] + buf[parity, r0 + u, pl.ds(k * 16, 16)]
                return tuple(accs)

            return lax.fori_loop(0, CHUNK // RUNROLL, row_body, accs)

        accs = tuple(jnp.zeros((16,), jnp.float32) for _ in range(NLG))
        accs = lax.fori_loop(0, NCHUNK, chunk_body, accs)
        for k in range(NLG):
            acc[t, pl.ds(k * 16, 16)] = accs[k]

    for t in range(4):
        pltpu.sync_copy(acc.at[t], out_hbm.at[t, wid])


def _finish_body(p_ref, r0_ref, w_ref, lmd_ref, out_ref):
    p = p_ref[...]
    r0 = r0_ref[...]
    inv = 1.0 / (VOCAB - 1)
    s = [
        jnp.sum(p[t * NW:(t + 1) * NW], axis=0, keepdims=True)
        - r0[t:t + 1]
        for t in range(4)
    ]
    du = (s[0] - s[2]) * inv
    dh = (s[1] - s[3]) * inv
    vec_sum = jnp.sum(du * du + dh * dh)
    w = w_ref[...]
    eye = (
        lax.broadcasted_iota(jnp.int32, (DIM, DIM), 0)
        == lax.broadcasted_iota(jnp.int32, (DIM, DIM), 1)
    ).astype(jnp.float32)
    wi = w - eye
    total = jnp.sum(wi * wi) + DIM * lmd_ref[0, 0] * vec_sum
    out_ref[...] = jnp.broadcast_to(total, (1, 1))


_finish = pl.pallas_call(
    _finish_body,
    out_shape=jax.ShapeDtypeStruct((1, 1), jnp.float32),
)


def kernel(emb1_u, emb1_h, emb2_u, emb2_h, W, lmd):
    partials = _colsums(emb1_u, emb1_h, emb2_u, emb2_h)
    row0 = jnp.stack([emb1_u[0], emb1_h[0], emb2_u[0], emb2_h[0]])
    lmd_arr = jnp.asarray(lmd, jnp.float32).reshape(1, 1)
    out = _finish(partials.reshape(4 * NW, DIM), row0, W, lmd_arr)
    return out[0, 0]
